# baseline (speedup 1.0000x reference)
import functools

import jax
import jax.numpy as jnp
from jax import lax
from jax.experimental import pallas as pl
from jax.experimental.pallas import tpu as pltpu
from jax.experimental.pallas import tpu_sc as plsc

_B = 128
_N = 32768
_NC = 2
_NS = 16
_L = 16
_NW = _NC * _NS
_ACC = 4

_SC_ROWS = _NW
_TC_ROWS = _B - _SC_ROWS

_CH = 4
_CHW = _N // _CH
_GPC = _CHW // (_ACC * _L)

_BR = 8
_INT_MAX = 2**31 - 1


def _argmax_sc_body(scores_hbm, out_hbm, buf, res_v, sem0, sem1):
    wid = lax.axis_index("c") * _NS + lax.axis_index("s")
    row = _TC_ROWS + wid
    lane = lax.iota(jnp.int32, _L)

    pltpu.make_async_copy(
        scores_hbm.at[row, pl.ds(0, _CHW)], buf.at[pl.ds(0, _CHW)], sem0
    ).start()
    pltpu.make_async_copy(
        scores_hbm.at[row, pl.ds(_CHW, _CHW)], buf.at[pl.ds(_CHW, _CHW)], sem1
    ).start()

    def chunk_body(c, carry):
        ms, cis = carry
        par = c & 1
        off = par * _CHW
        base = c * _GPC

        @pl.when(par == 0)
        def _():
            pltpu.make_async_copy(
                scores_hbm.at[row, pl.ds(0, _CHW)], buf.at[pl.ds(0, _CHW)],
                sem0,
            ).wait()

        @pl.when(par == 1)
        def _():
            pltpu.make_async_copy(
                scores_hbm.at[row, pl.ds(0, _CHW)], buf.at[pl.ds(0, _CHW)],
                sem1,
            ).wait()

        def body(g, carry2):
            ms2, cis2 = carry2
            gid = base + g
            new_ms, new_cis = [], []
            for a in range(_ACC):
                v = buf[pl.ds(off + g * (_ACC * _L) + a * _L, _L)]
                gt = v > ms2[a]
                new_ms.append(jnp.where(gt, v, ms2[a]))
                new_cis.append(jnp.where(gt, gid, cis2[a]))
            return tuple(new_ms), tuple(new_cis)

        ms, cis = lax.fori_loop(0, _GPC, body, (ms, cis), unroll=4)

        @pl.when((c < _CH - 2) & (par == 0))
        def _():
            pltpu.make_async_copy(
                scores_hbm.at[row, pl.ds((c + 2) * _CHW, _CHW)],
                buf.at[pl.ds(0, _CHW)], sem0,
            ).start()

        @pl.when((c < _CH - 2) & (par == 1))
        def _():
            pltpu.make_async_copy(
                scores_hbm.at[row, pl.ds((c + 2) * _CHW, _CHW)],
                buf.at[pl.ds(_CHW, _CHW)], sem1,
            ).start()

        return ms, cis

    m0 = tuple(jnp.full((_L,), -jnp.inf, jnp.float32) for _ in range(_ACC))
    i0 = tuple(jnp.zeros((_L,), jnp.int32) for _ in range(_ACC))
    ms, cis = lax.fori_loop(0, _CH, chunk_body, (m0, i0))

    m = ms[0]
    fi = cis[0] * (_ACC * _L) + lane
    for a in range(1, _ACC):
        qv = ms[a]
        qi = cis[a] * (_ACC * _L) + a * _L + lane
        take = (qv > m) | ((qv == m) & (qi < fi))
        m = jnp.where(take, qv, m)
        fi = jnp.where(take, qi, fi)

    for s in (8, 4, 2, 1):
        perm = lane ^ s
        qv = m.at[perm].get(mode="promise_in_bounds")
        qi = fi.at[perm].get(mode="promise_in_bounds")
        take = (qv > m) | ((qv == m) & (qi < fi))
        m = jnp.where(take, qv, m)
        fi = jnp.where(take, qi, fi)

    res_v[...] = fi
    pltpu.sync_copy(res_v, out_hbm.at[wid])


_argmax_sc = functools.partial(
    pl.kernel,
    out_type=jax.ShapeDtypeStruct((_NW, _L), jnp.int32),
    mesh=plsc.VectorSubcoreMesh(core_axis_name="c", subcore_axis_name="s"),
    scratch_types=[
        pltpu.VMEM((2 * _CHW,), jnp.float32),
        pltpu.VMEM((_L,), jnp.int32),
        pltpu.SemaphoreType.DMA,
        pltpu.SemaphoreType.DMA,
    ],
)(_argmax_sc_body)


def _argmax_tc_body(x_ref, o_ref):
    x = x_ref[...]
    m = jnp.max(x, axis=1, keepdims=True)
    iota = lax.broadcasted_iota(jnp.int32, (_BR, _N), 1)
    idx = jnp.min(jnp.where(x == m, iota, _INT_MAX), axis=1)
    o_ref[...] = idx.reshape(1, 1, _BR)


_argmax_tc = pl.pallas_call(
    _argmax_tc_body,
    grid=(_TC_ROWS // _BR,),
    in_specs=[pl.BlockSpec((_BR, _N), lambda i: (i, 0))],
    out_specs=pl.BlockSpec((1, 1, _BR), lambda i: (i, 0, 0)),
    out_shape=jax.ShapeDtypeStruct((_TC_ROWS // _BR, 1, _BR), jnp.int32),
)


@jax.jit
def kernel(scores):
    sc_out = _argmax_sc(scores)
    tc_out = _argmax_tc(scores)
    res = jnp.concatenate([tc_out.reshape(_TC_ROWS), sc_out[:, 0]])
    return res.astype(jnp.int64)


# speedup vs baseline: 1.0802x; 1.0802x over previous
import functools

import jax
import jax.numpy as jnp
from jax import lax
from jax.experimental import pallas as pl
from jax.experimental.pallas import tpu as pltpu
from jax.experimental.pallas import tpu_sc as plsc

_B = 128
_N = 32768
_NC = 2
_NS = 16
_L = 16
_ACC = 4

_SC_ROWS = 64
_TC_ROWS = _B - _SC_ROWS
_RPW = _SC_ROWS // (_NC * _NS)
_CHUNKS = _N // _L
_GROUPS = _CHUNKS // _ACC

_BR = 8
_INT_MAX = 2**31 - 1


def _argmax_sc_body(scores_hbm, out_hbm, buf, res_v, pack_v, out_v,
                    shared, sem0, sem1):
    cid = lax.axis_index("c")
    sid = lax.axis_index("s")
    lane = lax.iota(jnp.int32, _L)
    row0 = _TC_ROWS + cid * (_NS * _RPW) + sid * _RPW

    pltpu.make_async_copy(
        scores_hbm.at[row0], buf.at[pl.ds(0, _N)], sem0
    ).start()
    pltpu.make_async_copy(
        scores_hbm.at[row0 + 1], buf.at[pl.ds(_N, _N)], sem1
    ).start()

    def row_body(r, res):
        off = (r & 1) * _N

        @pl.when((r & 1) == 0)
        def _():
            pltpu.make_async_copy(
                scores_hbm.at[row0], buf.at[pl.ds(0, _N)], sem0
            ).wait()

        @pl.when((r & 1) == 1)
        def _():
            pltpu.make_async_copy(
                scores_hbm.at[row0], buf.at[pl.ds(0, _N)], sem1
            ).wait()

        def body(g, carry):
            ms, cis = carry
            new_ms, new_cis = [], []
            for a in range(_ACC):
                v = buf[pl.ds(off + g * (_ACC * _L) + a * _L, _L)]
                gt = v > ms[a]
                new_ms.append(jnp.where(gt, v, ms[a]))
                new_cis.append(jnp.where(gt, g, cis[a]))
            return tuple(new_ms), tuple(new_cis)

        m0 = tuple(jnp.full((_L,), -jnp.inf, jnp.float32) for _ in range(_ACC))
        i0 = tuple(jnp.zeros((_L,), jnp.int32) for _ in range(_ACC))
        ms, cis = lax.fori_loop(0, _GROUPS, body, (m0, i0), unroll=4)

        m = ms[0]
        fi = cis[0] * (_ACC * _L) + lane
        for a in range(1, _ACC):
            qv = ms[a]
            qi = cis[a] * (_ACC * _L) + a * _L + lane
            take = (qv > m) | ((qv == m) & (qi < fi))
            m = jnp.where(take, qv, m)
            fi = jnp.where(take, qi, fi)

        for s in (8, 4, 2, 1):
            perm = lane ^ s
            qv = m.at[perm].get(mode="promise_in_bounds")
            qi = fi.at[perm].get(mode="promise_in_bounds")
            take = (qv > m) | ((qv == m) & (qi < fi))
            m = jnp.where(take, qv, m)
            fi = jnp.where(take, qi, fi)

        return jnp.where(lane == (sid & 7) * _RPW + r, fi, res)

    res = lax.fori_loop(
        0, _RPW, row_body, jnp.zeros((_L,), jnp.int32), unroll=False
    )

    res_v[...] = res
    pltpu.sync_copy(res_v, shared.at[sid])
    plsc.subcore_barrier()

    @pl.when(sid == 0)
    def _():
        pltpu.sync_copy(shared, pack_v)
        for half in range(2):
            acc = pack_v[half * 8, :]
            for s in range(1, 8):
                acc = acc + pack_v[half * 8 + s, :]
            out_v[pl.ds(half * _L, _L)] = acc
        pltpu.sync_copy(out_v, out_hbm.at[cid])


_argmax_sc = functools.partial(
    pl.kernel,
    out_type=jax.ShapeDtypeStruct((_NC, _NS * _RPW), jnp.int32),
    mesh=plsc.VectorSubcoreMesh(core_axis_name="c", subcore_axis_name="s"),
    scratch_types=[
        pltpu.VMEM((2 * _N,), jnp.float32),
        pltpu.VMEM((_L,), jnp.int32),
        pltpu.VMEM((_NS, _L), jnp.int32),
        pltpu.VMEM((_NS * _RPW,), jnp.int32),
        pltpu.VMEM_SHARED((_NS, _L), jnp.int32),
        pltpu.SemaphoreType.DMA,
        pltpu.SemaphoreType.DMA,
    ],
)(_argmax_sc_body)


def _argmax_tc_body(x_ref, o_ref):
    x = x_ref[...]
    m = jnp.max(x, axis=1, keepdims=True)
    iota = lax.broadcasted_iota(jnp.int32, (_BR, _N), 1)
    idx = jnp.min(jnp.where(x == m, iota, _INT_MAX), axis=1)
    o_ref[...] = idx.reshape(1, 1, _BR)


_argmax_tc = pl.pallas_call(
    _argmax_tc_body,
    grid=(_TC_ROWS // _BR,),
    in_specs=[pl.BlockSpec((_BR, _N), lambda i: (i, 0))],
    out_specs=pl.BlockSpec((1, 1, _BR), lambda i: (i, 0, 0)),
    out_shape=jax.ShapeDtypeStruct((_TC_ROWS // _BR, 1, _BR), jnp.int32),
)


@jax.jit
def kernel(scores):
    sc_out = _argmax_sc(scores)
    tc_out = _argmax_tc(scores)
    res = jnp.concatenate([tc_out.reshape(_TC_ROWS), sc_out.reshape(_SC_ROWS)])
    return res.astype(jnp.int64)


# speedup vs baseline: 1.0869x; 1.0062x over previous
import functools

import jax
import jax.numpy as jnp
from jax import lax
from jax.experimental import pallas as pl
from jax.experimental.pallas import tpu as pltpu
from jax.experimental.pallas import tpu_sc as plsc

_B = 128
_N = 32768
_NC = 2
_NS = 16
_L = 16
_ACC = 4

_SC_ROWS = 64
_TC_ROWS = _B - _SC_ROWS
_RPW = _SC_ROWS // (_NC * _NS)
_CHUNKS = _N // _L
_GROUPS = _CHUNKS // _ACC

_BR = 8
_INT_MAX = 2**31 - 1


def _argmax_sc_body(scores_hbm, out_hbm, buf, res_v, pack_v, out_v,
                    shared, sem0, sem1):
    cid = lax.axis_index("c")
    sid = lax.axis_index("s")
    lane = lax.iota(jnp.int32, _L)
    row0 = _TC_ROWS + cid * (_NS * _RPW) + sid * _RPW

    pltpu.make_async_copy(
        scores_hbm.at[row0], buf.at[pl.ds(0, _N)], sem0
    ).start()
    pltpu.make_async_copy(
        scores_hbm.at[row0 + 1], buf.at[pl.ds(_N, _N)], sem1
    ).start()

    def row_body(r, res):
        off = (r & 1) * _N

        @pl.when((r & 1) == 0)
        def _():
            pltpu.make_async_copy(
                scores_hbm.at[row0], buf.at[pl.ds(0, _N)], sem0
            ).wait()

        @pl.when((r & 1) == 1)
        def _():
            pltpu.make_async_copy(
                scores_hbm.at[row0], buf.at[pl.ds(0, _N)], sem1
            ).wait()

        def body(g, carry):
            ms, cis = carry
            new_ms, new_cis = [], []
            for a in range(_ACC):
                v = buf[pl.ds(off + g * (_ACC * _L) + a * _L, _L)]
                gt = v > ms[a]
                new_ms.append(jnp.where(gt, v, ms[a]))
                new_cis.append(jnp.where(gt, g, cis[a]))
            return tuple(new_ms), tuple(new_cis)

        m0 = tuple(jnp.full((_L,), -jnp.inf, jnp.float32) for _ in range(_ACC))
        i0 = tuple(jnp.zeros((_L,), jnp.int32) for _ in range(_ACC))
        ms, cis = lax.fori_loop(0, _GROUPS, body, (m0, i0), unroll=4)

        m = ms[0]
        fi = cis[0] * (_ACC * _L) + lane
        for a in range(1, _ACC):
            qv = ms[a]
            qi = cis[a] * (_ACC * _L) + a * _L + lane
            take = (qv > m) | ((qv == m) & (qi < fi))
            m = jnp.where(take, qv, m)
            fi = jnp.where(take, qi, fi)

        for s in (8, 4, 2, 1):
            perm = lane ^ s
            qv = m.at[perm].get(mode="promise_in_bounds")
            qi = fi.at[perm].get(mode="promise_in_bounds")
            take = (qv > m) | ((qv == m) & (qi < fi))
            m = jnp.where(take, qv, m)
            fi = jnp.where(take, qi, fi)

        return jnp.where(lane == (sid & 7) * _RPW + r, fi, res)

    res = lax.fori_loop(
        0, _RPW, row_body, jnp.zeros((_L,), jnp.int32), unroll=False
    )

    res_v[...] = res
    pltpu.sync_copy(res_v, shared.at[pl.ds(sid * _L, _L)])
    plsc.subcore_barrier()

    @pl.when(sid == 0)
    def _():
        pltpu.sync_copy(shared, pack_v)
        for half in range(2):
            acc = pack_v[pl.ds((half * 8) * _L, _L)]
            for s in range(1, 8):
                acc = acc + pack_v[pl.ds((half * 8 + s) * _L, _L)]
            out_v[pl.ds(half * _L, _L)] = acc
        pltpu.sync_copy(out_v, out_hbm.at[cid])


_argmax_sc = functools.partial(
    pl.kernel,
    out_type=jax.ShapeDtypeStruct((_NC, _NS * _RPW), jnp.int32),
    mesh=plsc.VectorSubcoreMesh(core_axis_name="c", subcore_axis_name="s"),
    scratch_types=[
        pltpu.VMEM((2 * _N,), jnp.float32),
        pltpu.VMEM((_L,), jnp.int32),
        pltpu.VMEM((_NS * _L,), jnp.int32),
        pltpu.VMEM((_NS * _RPW,), jnp.int32),
        pltpu.VMEM_SHARED((_NS * _L,), jnp.int32),
        pltpu.SemaphoreType.DMA,
        pltpu.SemaphoreType.DMA,
    ],
)(_argmax_sc_body)


def _argmax_tc_body(x_ref, o_ref):
    x = x_ref[...]
    m = jnp.max(x, axis=1, keepdims=True)
    iota = lax.broadcasted_iota(jnp.int32, (_BR, _N), 1)
    idx = jnp.min(jnp.where(x == m, iota, _INT_MAX), axis=1)
    o_ref[...] = idx.reshape(1, 1, _BR)


_argmax_tc = pl.pallas_call(
    _argmax_tc_body,
    grid=(_TC_ROWS // _BR,),
    in_specs=[pl.BlockSpec((_BR, _N), lambda i: (i, 0))],
    out_specs=pl.BlockSpec((1, 1, _BR), lambda i: (i, 0, 0)),
    out_shape=jax.ShapeDtypeStruct((_TC_ROWS // _BR, 1, _BR), jnp.int32),
)


@jax.jit
def kernel(scores):
    sc_out = _argmax_sc(scores)
    tc_out = _argmax_tc(scores)
    res = jnp.concatenate([tc_out.reshape(_TC_ROWS), sc_out.reshape(_SC_ROWS)])
    return res.astype(jnp.int64)
